# Initial kernel scaffold; baseline (speedup 1.0000x reference)
import jax
import jax.numpy as jnp
from jax.experimental import pallas as pl


def kernel(ev, ej, ew, v_j, v_w, W_1, W_2, b, v):
    raise NotImplementedError("write your pallas kernel here")


# speedup vs baseline: 3.6904x; 3.6904x over previous
import functools

import jax
import jax.numpy as jnp
from jax import lax
from jax.experimental import pallas as pl
from jax.experimental.pallas import tpu as pltpu
from jax.experimental.pallas import tpu_sc as plsc


def _gather_body(consts, ejp, ewp, idxj, idxw, out,
                 idxj_v, idxw_v, rowsj_v, rowsw_v, semj, semw):
    (per_w, t_steps, c_rows, f_dim, dw_dim) = consts
    cid = lax.axis_index("c")
    sid = lax.axis_index("s")
    wid = sid * 2 + cid
    pltpu.sync_copy(idxj.at[wid], idxj_v)
    pltpu.sync_copy(idxw.at[wid], idxw_v)

    def step(t, carry):
        base = wid * per_w + t * c_rows
        cpj = pltpu.async_copy(ejp.at[idxj_v.at[t]], rowsj_v, semj)
        cpw = pltpu.async_copy(ewp.at[idxw_v.at[t]], rowsw_v, semw)
        cpj.wait()
        cpw.wait()
        pltpu.sync_copy(rowsj_v, out.at[pl.ds(base, c_rows), pl.ds(0, f_dim)])
        pltpu.sync_copy(rowsw_v, out.at[pl.ds(base, c_rows), pl.ds(f_dim, dw_dim)])
        return carry

    lax.fori_loop(0, t_steps, step, 0)


def _attn_body(bn, k, f_dim, gc_ref, ev_ref, w1a_ref, wcat_ref, b_ref, v_ref,
               out_ref):
    gc = gc_ref[...]
    base = jnp.dot(ev_ref[...], w1a_ref[...],
                   preferred_element_type=jnp.float32) + b_ref[...]
    av = jnp.dot(gc, wcat_ref[...], preferred_element_type=jnp.float32)
    a_dim = av.shape[-1]
    av3 = av.reshape(bn, k, a_dim) + base[:, None, :]
    r = jnp.maximum(av3, 0.0)
    x = jnp.sum(r * v_ref[...][None], axis=2)
    x = x - jnp.max(x, axis=1, keepdims=True)
    e = jnp.exp(x)
    a = e / jnp.sum(e, axis=1, keepdims=True)
    gj3 = gc[:, :f_dim].reshape(bn, k, f_dim)
    out_ref[...] = jnp.sum(a[:, :, None] * gj3, axis=1)


def kernel(ev, ej, ew, v_j, v_w, W_1, W_2, b, v):
    n, f_dim = ev.shape
    k = v_j.shape[1]
    dw_dim = ew.shape[1]
    a_dim = W_1.shape[1]
    e_rows = n * k
    fc = f_dim + dw_dim

    ej_p = jnp.concatenate([jnp.zeros((1, f_dim), ej.dtype), ej], axis=0)
    ew_p = jnp.concatenate([jnp.zeros((1, dw_dim), ew.dtype), ew], axis=0)
    w1a = W_1[:f_dim]
    wcat = jnp.concatenate([W_2, W_1[f_dim:]], axis=0)

    info = plsc.get_sparse_core_info()
    nw = info.num_cores * info.num_subcores
    assert e_rows % nw == 0
    per_w = e_rows // nw
    c_rows = 80
    assert per_w % c_rows == 0
    t_steps = per_w // c_rows
    idxj = v_j.reshape(nw, t_steps, c_rows)
    idxw = v_w.reshape(nw, t_steps, c_rows)

    mesh = plsc.VectorSubcoreMesh(core_axis_name="c", subcore_axis_name="s")
    gather = pl.kernel(
        functools.partial(_gather_body,
                          (per_w, t_steps, c_rows, f_dim, dw_dim)),
        out_type=jax.ShapeDtypeStruct((e_rows, fc), jnp.float32),
        mesh=mesh,
        scratch_types=[
            pltpu.VMEM((t_steps, c_rows), jnp.int32),
            pltpu.VMEM((t_steps, c_rows), jnp.int32),
            pltpu.VMEM((c_rows, f_dim), jnp.float32),
            pltpu.VMEM((c_rows, dw_dim), jnp.float32),
            pltpu.SemaphoreType.DMA,
            pltpu.SemaphoreType.DMA,
        ],
        compiler_params=pltpu.CompilerParams(use_tc_tiling_on_sc=False),
    )
    gc = gather(ej_p, ew_p, idxj, idxw)

    bn = 400
    assert n % bn == 0
    grid = (n // bn,)
    rb = bn * k
    attn = pl.pallas_call(
        functools.partial(_attn_body, bn, k, f_dim),
        grid=grid,
        in_specs=[
            pl.BlockSpec((rb, fc), lambda i: (i, 0)),
            pl.BlockSpec((bn, f_dim), lambda i: (i, 0)),
            pl.BlockSpec((f_dim, a_dim), lambda i: (0, 0)),
            pl.BlockSpec((fc, a_dim), lambda i: (0, 0)),
            pl.BlockSpec((1, a_dim), lambda i: (0, 0)),
            pl.BlockSpec((1, a_dim), lambda i: (0, 0)),
        ],
        out_specs=pl.BlockSpec((bn, f_dim), lambda i: (i, 0)),
        out_shape=jax.ShapeDtypeStruct((n, f_dim), jnp.float32),
    )
    return attn(gc, ev, w1a, wcat, b, v)


# speedup vs baseline: 4.7537x; 1.2881x over previous
import functools

import jax
import jax.numpy as jnp
from jax import lax
from jax.experimental import pallas as pl
from jax.experimental.pallas import tpu as pltpu
from jax.experimental.pallas import tpu_sc as plsc


def _gather_body(consts, ejp, ewp, idxj, idxw, outj, outw,
                 idxj_v, idxw_v, rowsj_v, rowsw_v, semj, semw):
    (per_w, t_steps, c_rows) = consts
    cid = lax.axis_index("c")
    sid = lax.axis_index("s")
    wid = sid * 2 + cid
    pltpu.sync_copy(idxj.at[wid], idxj_v)
    pltpu.sync_copy(idxw.at[wid], idxw_v)

    def step(t, carry):
        base = wid * per_w + t * c_rows
        cpj = pltpu.async_copy(ejp.at[idxj_v.at[t]], rowsj_v, semj)
        cpw = pltpu.async_copy(ewp.at[idxw_v.at[t]], rowsw_v, semw)
        cpj.wait()
        cpw.wait()
        pltpu.sync_copy(rowsj_v, outj.at[pl.ds(base, c_rows)])
        pltpu.sync_copy(rowsw_v, outw.at[pl.ds(base, c_rows)])
        return carry

    lax.fori_loop(0, t_steps, step, 0)


def _attn_body(bn, k, gj_ref, gw_ref, ev_ref, w1a_ref, w1b_ref, w2_ref,
               b_ref, v_ref, out_ref):
    gj = gj_ref[...]
    base = jnp.dot(ev_ref[...], w1a_ref[...],
                   preferred_element_type=jnp.float32) + b_ref[...]
    av = (jnp.dot(gj, w2_ref[...], preferred_element_type=jnp.float32) +
          jnp.dot(gw_ref[...], w1b_ref[...],
                  preferred_element_type=jnp.float32))
    a_dim = av.shape[-1]
    f_dim = gj.shape[-1]
    av3 = av.reshape(bn, k, a_dim) + base[:, None, :]
    r = jnp.maximum(av3, 0.0)
    x = jnp.sum(r * v_ref[...][None], axis=2)
    x = x - jnp.max(x, axis=1, keepdims=True)
    e = jnp.exp(x)
    a = e / jnp.sum(e, axis=1, keepdims=True)
    gj3 = gj.reshape(bn, k, f_dim)
    out_ref[...] = jnp.sum(a[:, :, None] * gj3, axis=1)


def kernel(ev, ej, ew, v_j, v_w, W_1, W_2, b, v):
    n, f_dim = ev.shape
    k = v_j.shape[1]
    dw_dim = ew.shape[1]
    a_dim = W_1.shape[1]
    e_rows = n * k

    ej_p = jnp.concatenate([jnp.zeros((1, f_dim), ej.dtype), ej], axis=0)
    ew_p = jnp.concatenate([jnp.zeros((1, dw_dim), ew.dtype), ew], axis=0)
    w1a = W_1[:f_dim]
    w1b = W_1[f_dim:]

    info = plsc.get_sparse_core_info()
    nw = info.num_cores * info.num_subcores
    assert e_rows % nw == 0
    per_w = e_rows // nw
    c_rows = 80
    assert per_w % c_rows == 0
    t_steps = per_w // c_rows
    idxj = v_j.reshape(nw, t_steps, c_rows)
    idxw = v_w.reshape(nw, t_steps, c_rows)

    mesh = plsc.VectorSubcoreMesh(core_axis_name="c", subcore_axis_name="s")
    gather = pl.kernel(
        functools.partial(_gather_body, (per_w, t_steps, c_rows)),
        out_type=[jax.ShapeDtypeStruct((e_rows, f_dim), jnp.float32),
                  jax.ShapeDtypeStruct((e_rows, dw_dim), jnp.float32)],
        mesh=mesh,
        scratch_types=[
            pltpu.VMEM((t_steps, c_rows), jnp.int32),
            pltpu.VMEM((t_steps, c_rows), jnp.int32),
            pltpu.VMEM((c_rows, f_dim), jnp.float32),
            pltpu.VMEM((c_rows, dw_dim), jnp.float32),
            pltpu.SemaphoreType.DMA,
            pltpu.SemaphoreType.DMA,
        ],
        compiler_params=pltpu.CompilerParams(use_tc_tiling_on_sc=False),
    )
    gj, gw = gather(ej_p, ew_p, idxj, idxw)

    bn = 400
    assert n % bn == 0
    grid = (n // bn,)
    rb = bn * k
    attn = pl.pallas_call(
        functools.partial(_attn_body, bn, k),
        grid=grid,
        in_specs=[
            pl.BlockSpec((rb, f_dim), lambda i: (i, 0)),
            pl.BlockSpec((rb, dw_dim), lambda i: (i, 0)),
            pl.BlockSpec((bn, f_dim), lambda i: (i, 0)),
            pl.BlockSpec((f_dim, a_dim), lambda i: (0, 0)),
            pl.BlockSpec((dw_dim, a_dim), lambda i: (0, 0)),
            pl.BlockSpec((f_dim, a_dim), lambda i: (0, 0)),
            pl.BlockSpec((1, a_dim), lambda i: (0, 0)),
            pl.BlockSpec((1, a_dim), lambda i: (0, 0)),
        ],
        out_specs=pl.BlockSpec((bn, f_dim), lambda i: (i, 0)),
        out_shape=jax.ShapeDtypeStruct((n, f_dim), jnp.float32),
    )
    return attn(gj, gw, ev, w1a, w1b, W_2, b, v)


# speedup vs baseline: 4.9634x; 1.0441x over previous
import functools

import jax
import jax.numpy as jnp
from jax import lax
from jax.experimental import pallas as pl
from jax.experimental.pallas import tpu as pltpu
from jax.experimental.pallas import tpu_sc as plsc


def _gather_body(consts, ej, ew, idxj, idxw, outj, outw,
                 idxj_v, idxw_v, rowsj_v, rowsw_v, semj, semw):
    (per_w, t_steps, c_rows) = consts
    cid = lax.axis_index("c")
    sid = lax.axis_index("s")
    wid = sid * 2 + cid
    pltpu.sync_copy(idxj.at[wid], idxj_v)
    pltpu.sync_copy(idxw.at[wid], idxw_v)

    def step(t, carry):
        base = wid * per_w + t * c_rows
        cpj = pltpu.async_copy(ej.at[idxj_v.at[t]], rowsj_v, semj)
        cpw = pltpu.async_copy(ew.at[idxw_v.at[t]], rowsw_v, semw)
        cpj.wait()
        cpw.wait()
        pltpu.sync_copy(rowsj_v, outj.at[pl.ds(base, c_rows)])
        pltpu.sync_copy(rowsw_v, outw.at[pl.ds(base, c_rows)])
        return carry

    lax.fori_loop(0, t_steps, step, 0)


def _attn_body(bn, k, gj_ref, gwp_ref, vj_ref, vw_ref, ev_ref, w1a_ref,
               w1b_ref, w2_ref, b_ref, v_ref, out_ref):
    gj = gj_ref[...]
    gwp = gwp_ref[...]
    dw = w1b_ref.shape[0]
    f_dim = gj.shape[-1]
    base = jnp.dot(ev_ref[...], w1a_ref[...],
                   preferred_element_type=jnp.float32) + b_ref[...]
    avj = jnp.dot(gj, w2_ref[...], preferred_element_type=jnp.float32)
    gw_cat = jnp.concatenate([gwp[:, g * dw:(g + 1) * dw] for g in range(8)],
                             axis=0)
    avw = jnp.dot(gw_cat, w1b_ref[...], preferred_element_type=jnp.float32)
    a_dim = avj.shape[-1]
    mj = (vj_ref[...] > 0).astype(jnp.float32)
    mw = (vw_ref[...] > 0).astype(jnp.float32)
    av3 = (base[:, None, :]
           + mj[:, :, None] * avj.reshape(bn, k, a_dim)
           + mw[:, :, None] * avw.reshape(bn, k, a_dim))
    r = jnp.maximum(av3, 0.0)
    x = jnp.sum(r * v_ref[...][None], axis=2)
    x = x - jnp.max(x, axis=1, keepdims=True)
    e = jnp.exp(x)
    a = e / jnp.sum(e, axis=1, keepdims=True)
    am = a * mj
    gj3 = gj.reshape(bn, k, f_dim)
    out_ref[...] = jnp.sum(am[:, :, None] * gj3, axis=1)


def kernel(ev, ej, ew, v_j, v_w, W_1, W_2, b, v):
    n, f_dim = ev.shape
    k = v_j.shape[1]
    dw_dim = ew.shape[1]
    a_dim = W_1.shape[1]
    e_rows = n * k
    pack = 128 // dw_dim

    w1a = W_1[:f_dim]
    w1b = W_1[f_dim:]
    bn = 400
    assert n % bn == 0
    nblk = n // bn
    rb = bn * k

    info = plsc.get_sparse_core_info()
    nw = info.num_cores * info.num_subcores
    assert e_rows % nw == 0
    per_w = e_rows // nw
    c_rows = 80
    assert per_w % c_rows == 0
    t_steps = per_w // c_rows

    jc = jnp.maximum(v_j - 1, 0)
    wc = jnp.maximum(v_w - 1, 0)
    idxj = jc.reshape(nw, t_steps, c_rows)
    idxw = (wc.reshape(nblk, pack, rb // pack)
            .transpose(0, 2, 1)
            .reshape(nw, t_steps, c_rows))

    mesh = plsc.VectorSubcoreMesh(core_axis_name="c", subcore_axis_name="s")
    gather = pl.kernel(
        functools.partial(_gather_body, (per_w, t_steps, c_rows)),
        out_type=[jax.ShapeDtypeStruct((e_rows, f_dim), jnp.float32),
                  jax.ShapeDtypeStruct((e_rows, dw_dim), jnp.float32)],
        mesh=mesh,
        scratch_types=[
            pltpu.VMEM((t_steps, c_rows), jnp.int32),
            pltpu.VMEM((t_steps, c_rows), jnp.int32),
            pltpu.VMEM((c_rows, f_dim), jnp.float32),
            pltpu.VMEM((c_rows, dw_dim), jnp.float32),
            pltpu.SemaphoreType.DMA,
            pltpu.SemaphoreType.DMA,
        ],
        compiler_params=pltpu.CompilerParams(use_tc_tiling_on_sc=False),
    )
    gj, gw = gather(ej, ew, idxj, idxw)
    gwp = gw.reshape(e_rows // pack, 128)

    grid = (nblk,)
    attn = pl.pallas_call(
        functools.partial(_attn_body, bn, k),
        grid=grid,
        in_specs=[
            pl.BlockSpec((rb, f_dim), lambda i: (i, 0)),
            pl.BlockSpec((rb // pack, 128), lambda i: (i, 0)),
            pl.BlockSpec((bn, k), lambda i: (i, 0)),
            pl.BlockSpec((bn, k), lambda i: (i, 0)),
            pl.BlockSpec((bn, f_dim), lambda i: (i, 0)),
            pl.BlockSpec((f_dim, a_dim), lambda i: (0, 0)),
            pl.BlockSpec((dw_dim, a_dim), lambda i: (0, 0)),
            pl.BlockSpec((f_dim, a_dim), lambda i: (0, 0)),
            pl.BlockSpec((1, a_dim), lambda i: (0, 0)),
            pl.BlockSpec((1, a_dim), lambda i: (0, 0)),
        ],
        out_specs=pl.BlockSpec((bn, f_dim), lambda i: (i, 0)),
        out_shape=jax.ShapeDtypeStruct((n, f_dim), jnp.float32),
    )
    return attn(gj, gwp, v_j, v_w, ev, w1a, w1b, W_2, b, v)


# speedup vs baseline: 5.0834x; 1.0242x over previous
import functools

import jax
import jax.numpy as jnp
import numpy as np
from jax import lax
from jax.experimental import pallas as pl
from jax.experimental.pallas import tpu as pltpu
from jax.experimental.pallas import tpu_sc as plsc


def _gather_body(consts, ej, ew, idxj, idxw, outj, outw,
                 idxj_v, idxw_v, rowsj_v, rowsw_v, semj, semw):
    (per_w, t_steps, c_rows) = consts
    cid = lax.axis_index("c")
    sid = lax.axis_index("s")
    wid = sid * 2 + cid
    pltpu.sync_copy(idxj.at[wid], idxj_v)
    pltpu.sync_copy(idxw.at[wid], idxw_v)

    def step(t, carry):
        base = wid * per_w + t * c_rows
        cpj = pltpu.async_copy(ej.at[idxj_v.at[t]], rowsj_v, semj)
        cpw = pltpu.async_copy(ew.at[idxw_v.at[t]], rowsw_v, semw)
        cpj.wait()
        cpw.wait()
        pltpu.sync_copy(rowsj_v, outj.at[pl.ds(base, c_rows)])
        pltpu.sync_copy(rowsw_v, outw.at[pl.ds(base, c_rows)])
        return carry

    lax.fori_loop(0, t_steps, step, 0)


def _attn_body(bn, k, gj_ref, gwp_ref, vj_ref, vw_ref, ev_ref, w1a_ref,
               w1b_ref, w2_ref, b_ref, v_ref, out_ref):
    gj = gj_ref[...]
    gwp = gwp_ref[...]
    dw = w1b_ref.shape[0]
    f_dim = gj.shape[-1]
    base = jnp.dot(ev_ref[...], w1a_ref[...],
                   preferred_element_type=jnp.float32) + b_ref[...]
    avj = jnp.dot(gj, w2_ref[...], preferred_element_type=jnp.float32)
    avw = jnp.concatenate(
        [jnp.dot(gwp[:, g * dw:(g + 1) * dw], w1b_ref[...],
                 preferred_element_type=jnp.float32) for g in range(8)],
        axis=0)
    a_dim = avj.shape[-1]
    mj = (vj_ref[...] > 0).astype(jnp.float32)
    mw = (vw_ref[...] > 0).astype(jnp.float32)
    av3 = (base[:, None, :]
           + mj[:, :, None] * avj.reshape(bn, k, a_dim)
           + mw[:, :, None] * avw.reshape(bn, k, a_dim))
    r = jnp.maximum(av3, 0.0)
    x = jnp.sum(r * v_ref[...][None], axis=2)
    x = x - jnp.max(x, axis=1, keepdims=True)
    e = jnp.exp(x)
    a = e / jnp.sum(e, axis=1, keepdims=True)
    am = a * mj
    gj3 = gj.reshape(bn, k, f_dim)
    out_ref[...] = jnp.sum(am[:, :, None] * gj3, axis=1)


def kernel(ev, ej, ew, v_j, v_w, W_1, W_2, b, v):
    n, f_dim = ev.shape
    k = v_j.shape[1]
    dw_dim = ew.shape[1]
    a_dim = W_1.shape[1]
    e_rows = n * k
    pack = 128 // dw_dim

    w1a = W_1[:f_dim]
    w1b = W_1[f_dim:]
    bn = 400
    assert n % bn == 0
    nblk = n // bn
    rb = bn * k

    info = plsc.get_sparse_core_info()
    nw = info.num_cores * info.num_subcores
    assert e_rows % nw == 0
    per_w = e_rows // nw
    c_rows = 80
    assert per_w % c_rows == 0
    t_steps = per_w // c_rows

    jc = jnp.maximum(v_j - 1, 0)
    wc = jnp.maximum(v_w - 1, 0)
    idxj = jc.reshape(nw, t_steps, c_rows)
    q = np.arange(e_rows, dtype=np.int32)
    blk0 = (q // rb) * rb
    qq = q - blk0
    perm = blk0 + (rb // pack) * (qq % pack) + qq // pack
    idxw = jnp.take(wc.reshape(-1), jnp.asarray(perm),
                    axis=0).reshape(nw, t_steps, c_rows)

    mesh = plsc.VectorSubcoreMesh(core_axis_name="c", subcore_axis_name="s")
    gather = pl.kernel(
        functools.partial(_gather_body, (per_w, t_steps, c_rows)),
        out_type=[jax.ShapeDtypeStruct((e_rows, f_dim), jnp.float32),
                  jax.ShapeDtypeStruct((e_rows, dw_dim), jnp.float32)],
        mesh=mesh,
        scratch_types=[
            pltpu.VMEM((t_steps, c_rows), jnp.int32),
            pltpu.VMEM((t_steps, c_rows), jnp.int32),
            pltpu.VMEM((c_rows, f_dim), jnp.float32),
            pltpu.VMEM((c_rows, dw_dim), jnp.float32),
            pltpu.SemaphoreType.DMA,
            pltpu.SemaphoreType.DMA,
        ],
        compiler_params=pltpu.CompilerParams(use_tc_tiling_on_sc=False),
    )
    gj, gw = gather(ej, ew, idxj, idxw)
    gwp = gw.reshape(e_rows // pack, 128)

    grid = (nblk,)
    attn = pl.pallas_call(
        functools.partial(_attn_body, bn, k),
        grid=grid,
        in_specs=[
            pl.BlockSpec((rb, f_dim), lambda i: (i, 0)),
            pl.BlockSpec((rb // pack, 128), lambda i: (i, 0)),
            pl.BlockSpec((bn, k), lambda i: (i, 0)),
            pl.BlockSpec((bn, k), lambda i: (i, 0)),
            pl.BlockSpec((bn, f_dim), lambda i: (i, 0)),
            pl.BlockSpec((f_dim, a_dim), lambda i: (0, 0)),
            pl.BlockSpec((dw_dim, a_dim), lambda i: (0, 0)),
            pl.BlockSpec((f_dim, a_dim), lambda i: (0, 0)),
            pl.BlockSpec((1, a_dim), lambda i: (0, 0)),
            pl.BlockSpec((1, a_dim), lambda i: (0, 0)),
        ],
        out_specs=pl.BlockSpec((bn, f_dim), lambda i: (i, 0)),
        out_shape=jax.ShapeDtypeStruct((n, f_dim), jnp.float32),
    )
    return attn(gj, gwp, v_j, v_w, ev, w1a, w1b, W_2, b, v)


# speedup vs baseline: 6.4679x; 1.2724x over previous
import functools

import jax
import jax.numpy as jnp
import numpy as np
from jax import lax
from jax.experimental import pallas as pl
from jax.experimental.pallas import tpu as pltpu
from jax.experimental.pallas import tpu_sc as plsc


def _gather_body(consts, ej, ew, idxj, idxw, outj, outw,
                 idxj_v, idxw_v, rowsj_v, rowsw_v, semj, semw):
    (per_w, t_steps, c_rows) = consts
    cid = lax.axis_index("c")
    sid = lax.axis_index("s")
    wid = sid * 2 + cid
    pltpu.sync_copy(idxj.at[wid], idxj_v)
    pltpu.sync_copy(idxw.at[wid], idxw_v)

    def step(t, carry):
        base = wid * per_w + t * c_rows
        cpj = pltpu.async_copy(ej.at[idxj_v.at[t]], rowsj_v, semj)
        cpw = pltpu.async_copy(ew.at[idxw_v.at[t]], rowsw_v, semw)
        cpj.wait()
        cpw.wait()
        pltpu.sync_copy(rowsj_v, outj.at[pl.ds(base, c_rows)])
        pltpu.sync_copy(rowsw_v, outw.at[pl.ds(base, c_rows)])
        return carry

    lax.fori_loop(0, t_steps, step, 0)


def _attn_body(bn, k, gj_ref, gwp_ref, vj_ref, vw_ref, ev_ref, w1a_ref,
               w1b_ref, w2_ref, b_ref, v_ref, out_ref):
    gj = gj_ref[...]
    gwp = gwp_ref[...]
    dw = w1b_ref.shape[0]
    f_dim = gj.shape[-1]
    base = jnp.dot(ev_ref[...], w1a_ref[...],
                   preferred_element_type=jnp.float32) + b_ref[...]
    avj = jnp.dot(gj, w2_ref[...], preferred_element_type=jnp.float32)
    avw = jnp.concatenate(
        [jnp.dot(gwp[:, g * dw:(g + 1) * dw], w1b_ref[...],
                 preferred_element_type=jnp.float32) for g in range(8)],
        axis=0)
    a_dim = avj.shape[-1]
    mj = (vj_ref[...] > 0).astype(jnp.float32)
    mw = (vw_ref[...] > 0).astype(jnp.float32)
    av3 = (base[:, None, :]
           + mj[:, :, None] * avj.reshape(bn, k, a_dim)
           + mw[:, :, None] * avw.reshape(bn, k, a_dim))
    r = jnp.maximum(av3, 0.0)
    x = jnp.sum(r * v_ref[...][None], axis=2)
    x = x - jnp.max(x, axis=1, keepdims=True)
    e = jnp.exp(x)
    a = e / jnp.sum(e, axis=1, keepdims=True)
    am = a * mj
    gj3 = gj.reshape(bn, k, f_dim)
    out_ref[...] = jnp.sum(am[:, :, None] * gj3, axis=1)


def kernel(ev, ej, ew, v_j, v_w, W_1, W_2, b, v):
    n, f_dim = ev.shape
    k = v_j.shape[1]
    dw_dim = ew.shape[1]
    a_dim = W_1.shape[1]
    e_rows = n * k
    pack = 128 // dw_dim

    w1a = W_1[:f_dim]
    w1b = W_1[f_dim:]
    bn = 400
    assert n % bn == 0
    nblk = n // bn
    rb = bn * k

    info = plsc.get_sparse_core_info()
    nw = info.num_cores * info.num_subcores
    n_chunks = 5
    nc = n // n_chunks
    ec = nc * k
    assert ec % nw == 0
    per_w = ec // nw
    c_rows = 80
    assert per_w % c_rows == 0
    t_steps = per_w // c_rows
    assert nc % bn == 0
    nblk_c = nc // bn

    jc = jnp.maximum(v_j - 1, 0)
    wc = jnp.maximum(v_w - 1, 0)
    wcp = jnp.transpose(wc.reshape(n // bn, pack, rb // pack), (0, 2, 1))

    mesh = plsc.VectorSubcoreMesh(core_axis_name="c", subcore_axis_name="s")
    gather = pl.kernel(
        functools.partial(_gather_body, (per_w, t_steps, c_rows)),
        out_type=[jax.ShapeDtypeStruct((ec, f_dim), jnp.float32),
                  jax.ShapeDtypeStruct((ec, dw_dim), jnp.float32)],
        mesh=mesh,
        scratch_types=[
            pltpu.VMEM((t_steps, c_rows), jnp.int32),
            pltpu.VMEM((t_steps, c_rows), jnp.int32),
            pltpu.VMEM((c_rows, f_dim), jnp.float32),
            pltpu.VMEM((c_rows, dw_dim), jnp.float32),
            pltpu.SemaphoreType.DMA,
            pltpu.SemaphoreType.DMA,
        ],
        compiler_params=pltpu.CompilerParams(use_tc_tiling_on_sc=False),
    )

    attn = pl.pallas_call(
        functools.partial(_attn_body, bn, k),
        grid=(nblk_c,),
        in_specs=[
            pl.BlockSpec((rb, f_dim), lambda i: (i, 0)),
            pl.BlockSpec((rb // pack, 128), lambda i: (i, 0)),
            pl.BlockSpec((bn, k), lambda i: (i, 0)),
            pl.BlockSpec((bn, k), lambda i: (i, 0)),
            pl.BlockSpec((bn, f_dim), lambda i: (i, 0)),
            pl.BlockSpec((f_dim, a_dim), lambda i: (0, 0)),
            pl.BlockSpec((dw_dim, a_dim), lambda i: (0, 0)),
            pl.BlockSpec((f_dim, a_dim), lambda i: (0, 0)),
            pl.BlockSpec((1, a_dim), lambda i: (0, 0)),
            pl.BlockSpec((1, a_dim), lambda i: (0, 0)),
        ],
        out_specs=pl.BlockSpec((bn, f_dim), lambda i: (i, 0)),
        out_shape=jax.ShapeDtypeStruct((nc, f_dim), jnp.float32),
    )

    outs = []
    for s in range(n_chunks):
        nd = slice(s * nc, (s + 1) * nc)
        idxj_s = jc[nd].reshape(nw, t_steps, c_rows)
        idxw_s = wcp[s * nblk_c:(s + 1) * nblk_c].reshape(nw, t_steps, c_rows)
        gj, gw = gather(ej, ew, idxj_s, idxw_s)
        gwp = gw.reshape(ec // pack, 128)
        outs.append(attn(gj, gwp, v_j[nd], v_w[nd], ev[nd],
                         w1a, w1b, W_2, b, v))
    return jnp.concatenate(outs, axis=0)
